# baseline (speedup 1.0000x reference)
import math

import jax
import jax.numpy as jnp
import numpy as np
from jax.experimental import pallas as pl
from jax.experimental.pallas import tpu as pltpu

LB = 256


def _conv1_placement():
    T = np.zeros((25, 8, 12, 224), np.float32)
    for pj in range(2):
        for rw in range(2):
            for rp in range(2):
                P = pj * 4 + rw * 2 + rp
                ri = 2 * rp + rw
                for joh in range(12):
                    jo = 2 * joh + pj
                    for di in range(5):
                        for dj in range(5):
                            T[di * 5 + dj, P, joh, (ri + di) * 28 + jo + dj] = 1.0
    return T


def _conv2_placement():
    T = np.zeros((251, 4, 4, 768), np.float32)
    for psj in range(2):
        for s in range(2):
            P = psj * 2 + s
            for sjh in range(4):
                sj = 2 * sjh + psj
                for ci in range(10):
                    for di in range(5):
                        for dj in range(5):
                            j = sj + dj
                            if j < 12:
                                T[ci * 25 + di * 5 + dj, P, sjh,
                                  (s + di) * 128 + ci * 12 + j] = 1.0
                T[250, P, sjh, s * 128 + 120] = 1.0
    return T


_T1 = _conv1_placement()
_T2 = _conv2_placement()
_PF1 = np.zeros((400, 512), np.float32)
for _q in range(4):
    for _c in range(20):
        for _w in range(4):
            _PF1[_q * 80 + _c * 4 + _w, (_q * 4 + _w) * 32 + _c] = 1.0
_B1OH = np.zeros((240, 16), np.float32)
for _r in range(240):
    _B1OH[_r, (_r // 12) % 10] = 1.0
_BH1 = np.zeros((400,), np.float32)
_BH1[320] = 1.0


def _lenet_kernel(x_ref, w1_ref, b1_ref, w2_ref, f1_ref, f2_ref, bf2_ref, o_ref,
                  p1, p2):

    p1[:, 120:128, :] = jnp.ones((12, 8, LB), jnp.float32)
    p2[4] = jnp.ones((80, LB), jnp.float32)

    w1 = w1_ref[...]
    for t in range(6):
        a = x_ref[pl.ds(4 * t, 8)].reshape(224, LB)
        r = jnp.dot(w1, a, preferred_element_type=jnp.float32)
        r = r.reshape(2, 2, 240, LB)
        m = jnp.maximum(r[0], r[1])
        m = jnp.maximum(m[0], m[1]) + b1_ref[...]
        p1[pl.ds(2 * t, 2), 0:120, :] = jnp.maximum(m, 0.0).reshape(2, 120, LB)

    w2 = w2_ref[...]
    for q in range(4):
        a = p1[pl.ds(2 * q, 6)].reshape(768, LB)
        r = jnp.dot(w2, a, preferred_element_type=jnp.float32)
        r = r.reshape(2, 2, 80, LB)
        m = jnp.maximum(r[0], r[1])
        m = jnp.maximum(m[0], m[1])
        p2[q] = jnp.maximum(m, 0.0)

    a3 = p2[...].reshape(400, LB)
    h = jnp.dot(f1_ref[...], a3, preferred_element_type=jnp.float32)
    h = jnp.maximum(h, 0.0)
    logits = jnp.dot(f2_ref[...], h,
                     preferred_element_type=jnp.float32) + bf2_ref[...]
    mx = jnp.max(logits, axis=0, keepdims=True)
    sh = logits - mx
    lse = jnp.log(jnp.sum(jnp.exp(sh), axis=0, keepdims=True))
    ls = sh - lse
    o_ref[...] = ls.T[:, 0:10]


def kernel(x, w1t, b1t, w2blk, b2t, wf1p, bf1p, wf2p, bf2p):
    N = x.shape[0]
    G = (N + LB - 1) // LB
    Npad = G * LB

    xi = x.reshape(N, 784)
    if Npad != N:
        xi = jnp.pad(xi, ((0, Npad - N), (0, 0)))
    xp = xi.T.reshape(28, 28, Npad)

    w1aug = jnp.transpose(w1t[:25, :10])
    b1r = jnp.broadcast_to(jnp.dot(_B1OH, b1t[0, :16])[:, None], (240, LB))
    w2aug = jnp.concatenate(
        [jnp.transpose(w2blk[:, :10, :20], (2, 1, 0)).reshape(20, 250),
         b2t[:1, :20].T], axis=1)
    W1 = jnp.einsum('ct,tPJk->PcJk', w1aug, _T1).reshape(960, 224)
    W2 = jnp.einsum('ct,tPJk->PcJk', w2aug, _T2).reshape(320, 768)
    F1 = (jnp.dot(_PF1, wf1p).T
          + bf1p.reshape(64, 1) * _BH1[None, :])
    F2 = jnp.transpose(wf2p[:, :16])
    bf2v = jnp.broadcast_to(bf2p[0, :16].reshape(16, 1), (16, LB))

    flops = G * 2 * LB * (6 * 960 * 224 + 4 * 320 * 768 + 64 * 400 + 16 * 64)
    bytes_accessed = 4 * (Npad * (28 * 28 + 10)
                          + 960 * 256 + 320 * 768 + 64 * 400 + 16 * 64)
    out = pl.pallas_call(
        _lenet_kernel,
        out_shape=jax.ShapeDtypeStruct((Npad, 10), jnp.float32),
        grid_spec=pltpu.PrefetchScalarGridSpec(
            num_scalar_prefetch=0,
            grid=(G,),
            in_specs=[
                pl.BlockSpec((28, 28, LB), lambda i: (0, 0, i)),
                pl.BlockSpec((960, 224), lambda i: (0, 0)),
                pl.BlockSpec((240, LB), lambda i: (0, 0)),
                pl.BlockSpec((320, 768), lambda i: (0, 0)),
                pl.BlockSpec((64, 400), lambda i: (0, 0)),
                pl.BlockSpec((16, 64), lambda i: (0, 0)),
                pl.BlockSpec((16, LB), lambda i: (0, 0)),
            ],
            out_specs=pl.BlockSpec((LB, 10), lambda i: (i, 0)),
            scratch_shapes=[
                pltpu.VMEM((12, 128, LB), jnp.float32),
                pltpu.VMEM((5, 80, LB), jnp.float32),
            ],
        ),
        compiler_params=pltpu.CompilerParams(
            dimension_semantics=("parallel",),
            vmem_limit_bytes=32 * 1024 * 1024,
        ),
        cost_estimate=pl.CostEstimate(
            flops=flops, transcendentals=N * 17,
            bytes_accessed=bytes_accessed),
    )(xp, W1, b1r, W2, F1, F2, bf2v)
    return out[:N] if Npad != N else out


# speedup vs baseline: 1.6348x; 1.6348x over previous
import math

import jax
import jax.numpy as jnp
import numpy as np
from jax.experimental import pallas as pl
from jax.experimental.pallas import tpu as pltpu

LB = 512


def _conv1_placement():
    T = np.zeros((26, 8, 12, 256), np.float32)
    for pj in range(2):
        for rw in range(2):
            for rp in range(2):
                P = pj * 4 + rw * 2 + rp
                ri = 2 * rp + rw
                for joh in range(12):
                    jo = 2 * joh + pj
                    for di in range(5):
                        for dj in range(5):
                            j = jo + dj
                            if j < 28:
                                T[di * 5 + dj, P, joh, (ri + di) * 32 + j] = 1.0
                    T[25, P, joh, ri * 32 + 28] = 1.0
    return T


def _conv2_placement():
    T = np.zeros((251, 4, 4, 768), np.float32)
    for psj in range(2):
        for s in range(2):
            P = psj * 2 + s
            for sjh in range(4):
                sj = 2 * sjh + psj
                for ci in range(10):
                    for di in range(5):
                        for dj in range(5):
                            j = sj + dj
                            if j < 12:
                                T[ci * 25 + di * 5 + dj, P, sjh,
                                  (s + di) * 128 + ci * 12 + j] = 1.0
                T[250, P, sjh, s * 128 + 120] = 1.0
    return T


_T1 = _conv1_placement()
_T2 = _conv2_placement()
_PF1 = np.zeros((400, 512), np.float32)
for _q in range(4):
    for _c in range(20):
        for _w in range(4):
            _PF1[_q * 80 + _c * 4 + _w, (_q * 4 + _w) * 32 + _c] = 1.0
_BH1 = np.zeros((400,), np.float32)
_BH1[320] = 1.0


def _lenet_kernel(x_ref, w1_ref, w2_ref, f1_ref, f2_ref, bf2_ref, o_ref,
                  p1, p2):

    p1[:, 120:128, :] = jnp.ones((12, 8, LB), jnp.float32)
    p2[4] = jnp.ones((80, LB), jnp.float32)

    w1 = w1_ref[...]
    for t in range(6):
        a = x_ref[pl.ds(4 * t, 8)].reshape(256, LB)
        r = jnp.dot(w1, a, preferred_element_type=jnp.float32)
        r = r.reshape(2, 2, 240, LB)
        m = jnp.maximum(r[0], r[1])
        m = jnp.maximum(m[0], m[1])
        p1[pl.ds(2 * t, 2), 0:120, :] = jnp.maximum(m, 0.0).reshape(2, 120, LB)

    w2 = w2_ref[...]
    for q in range(4):
        a = p1[pl.ds(2 * q, 6)].reshape(768, LB)
        r = jnp.dot(w2, a, preferred_element_type=jnp.float32)
        r = r.reshape(2, 2, 80, LB)
        m = jnp.maximum(r[0], r[1])
        m = jnp.maximum(m[0], m[1])
        p2[q] = jnp.maximum(m, 0.0)

    a3 = p2[...].reshape(400, LB)
    h = jnp.dot(f1_ref[...], a3, preferred_element_type=jnp.float32)
    h = jnp.maximum(h, 0.0)
    logits = jnp.dot(f2_ref[...], h,
                     preferred_element_type=jnp.float32) + bf2_ref[...]
    mx = jnp.max(logits, axis=0, keepdims=True)
    sh = logits - mx
    lse = jnp.log(jnp.sum(jnp.exp(sh), axis=0, keepdims=True))
    ls = sh - lse
    o_ref[...] = ls.T[:, 0:10]


def kernel(x, w1t, b1t, w2blk, b2t, wf1p, bf1p, wf2p, bf2p):
    N = x.shape[0]
    G = (N + LB - 1) // LB
    Npad = G * LB

    xi = jnp.pad(x.reshape(N, 28, 28), ((0, Npad - N), (0, 0), (0, 4)),
                 constant_values=1.0).reshape(Npad, 896)
    xp = xi.T.reshape(28, 32, Npad)

    w1aug = jnp.concatenate([jnp.transpose(w1t[:25, :10]),
                             b1t[:1, :10].T], axis=1)
    w2aug = jnp.concatenate(
        [jnp.transpose(w2blk[:, :10, :20], (2, 1, 0)).reshape(20, 250),
         b2t[:1, :20].T], axis=1)
    W1 = jnp.einsum('ct,tPJk->PcJk', w1aug, _T1).reshape(960, 256)
    W2 = jnp.einsum('ct,tPJk->PcJk', w2aug, _T2).reshape(320, 768)
    F1 = (jnp.dot(_PF1, wf1p).T
          + bf1p.reshape(64, 1) * _BH1[None, :])
    F2 = jnp.transpose(wf2p[:, :16])
    bf2v = jnp.broadcast_to(bf2p[0, :16].reshape(16, 1), (16, LB))

    flops = G * 2 * LB * (6 * 960 * 256 + 4 * 320 * 768 + 64 * 400 + 16 * 64)
    bytes_accessed = 4 * (Npad * (28 * 32 + 10)
                          + 960 * 256 + 320 * 768 + 64 * 400 + 16 * 64)
    out = pl.pallas_call(
        _lenet_kernel,
        out_shape=jax.ShapeDtypeStruct((Npad, 10), jnp.float32),
        grid_spec=pltpu.PrefetchScalarGridSpec(
            num_scalar_prefetch=0,
            grid=(G,),
            in_specs=[
                pl.BlockSpec((28, 32, LB), lambda i: (0, 0, i)),
                pl.BlockSpec((960, 256), lambda i: (0, 0)),
                pl.BlockSpec((320, 768), lambda i: (0, 0)),
                pl.BlockSpec((64, 400), lambda i: (0, 0)),
                pl.BlockSpec((16, 64), lambda i: (0, 0)),
                pl.BlockSpec((16, LB), lambda i: (0, 0)),
            ],
            out_specs=pl.BlockSpec((LB, 10), lambda i: (i, 0)),
            scratch_shapes=[
                pltpu.VMEM((12, 128, LB), jnp.float32),
                pltpu.VMEM((5, 80, LB), jnp.float32),
            ],
        ),
        compiler_params=pltpu.CompilerParams(
            dimension_semantics=("parallel",),
            vmem_limit_bytes=32 * 1024 * 1024,
        ),
        cost_estimate=pl.CostEstimate(
            flops=flops, transcendentals=N * 17,
            bytes_accessed=bytes_accessed),
    )(xp, W1, W2, F1, F2, bf2v)
    return out[:N] if Npad != N else out


# speedup vs baseline: 1.7159x; 1.0496x over previous
import math

import jax
import jax.numpy as jnp
import numpy as np
from jax.experimental import pallas as pl
from jax.experimental.pallas import tpu as pltpu

LB = 1024


def _conv1_placement():
    T = np.zeros((26, 8, 12, 256), np.float32)
    for pj in range(2):
        for rw in range(2):
            for rp in range(2):
                P = pj * 4 + rw * 2 + rp
                ri = 2 * rp + rw
                for joh in range(12):
                    jo = 2 * joh + pj
                    for di in range(5):
                        for dj in range(5):
                            j = jo + dj
                            if j < 28:
                                T[di * 5 + dj, P, joh, (ri + di) * 32 + j] = 1.0
                    T[25, P, joh, ri * 32 + 28] = 1.0
    return T


def _conv2_placement():
    T = np.zeros((251, 4, 4, 768), np.float32)
    for psj in range(2):
        for s in range(2):
            P = psj * 2 + s
            for sjh in range(4):
                sj = 2 * sjh + psj
                for ci in range(10):
                    for di in range(5):
                        for dj in range(5):
                            j = sj + dj
                            if j < 12:
                                T[ci * 25 + di * 5 + dj, P, sjh,
                                  (s + di) * 128 + ci * 12 + j] = 1.0
                T[250, P, sjh, s * 128 + 120] = 1.0
    return T


_T1 = _conv1_placement()
_T2 = _conv2_placement()
_PF1 = np.zeros((400, 512), np.float32)
for _q in range(4):
    for _c in range(20):
        for _w in range(4):
            _PF1[_q * 80 + _c * 4 + _w, (_q * 4 + _w) * 32 + _c] = 1.0
_BH1 = np.zeros((400,), np.float32)
_BH1[320] = 1.0


def _lenet_kernel(x_ref, w1_ref, w2_ref, f1_ref, f2_ref, bf2_ref, o_ref,
                  p1, p2):

    p1[:, 120:128, :] = jnp.ones((12, 8, LB), jnp.float32)
    p2[4] = jnp.ones((80, LB), jnp.float32)

    w1 = w1_ref[...]
    for t in range(6):
        a = x_ref[pl.ds(4 * t, 8)].reshape(256, LB)
        r = jnp.dot(w1, a, preferred_element_type=jnp.float32)
        r = r.reshape(2, 2, 240, LB)
        m = jnp.maximum(r[0], r[1])
        m = jnp.maximum(m[0], m[1])
        p1[pl.ds(2 * t, 2), 0:120, :] = jnp.maximum(m, 0.0).reshape(2, 120, LB)

    w2 = w2_ref[...]
    for q in range(4):
        a = p1[pl.ds(2 * q, 6)].reshape(768, LB)
        r = jnp.dot(w2, a, preferred_element_type=jnp.float32)
        r = r.reshape(2, 2, 80, LB)
        m = jnp.maximum(r[0], r[1])
        m = jnp.maximum(m[0], m[1])
        p2[q] = jnp.maximum(m, 0.0)

    a3 = p2[...].reshape(400, LB)
    h = jnp.dot(f1_ref[...], a3, preferred_element_type=jnp.float32)
    h = jnp.maximum(h, 0.0)
    logits = jnp.dot(f2_ref[...], h,
                     preferred_element_type=jnp.float32) + bf2_ref[...]
    mx = jnp.max(logits, axis=0, keepdims=True)
    sh = logits - mx
    lse = jnp.log(jnp.sum(jnp.exp(sh), axis=0, keepdims=True))
    ls = sh - lse
    o_ref[...] = ls.T[:, 0:10]


def kernel(x, w1t, b1t, w2blk, b2t, wf1p, bf1p, wf2p, bf2p):
    N = x.shape[0]
    G = (N + LB - 1) // LB
    Npad = G * LB

    xi = jnp.pad(x.reshape(N, 28, 28), ((0, Npad - N), (0, 0), (0, 4)),
                 constant_values=1.0).reshape(Npad, 896)
    xp = xi.T.reshape(28, 32, Npad)

    w1aug = jnp.concatenate([jnp.transpose(w1t[:25, :10]),
                             b1t[:1, :10].T], axis=1)
    w2aug = jnp.concatenate(
        [jnp.transpose(w2blk[:, :10, :20], (2, 1, 0)).reshape(20, 250),
         b2t[:1, :20].T], axis=1)
    W1 = jnp.einsum('ct,tPJk->PcJk', w1aug, _T1).reshape(960, 256)
    W2 = jnp.einsum('ct,tPJk->PcJk', w2aug, _T2).reshape(320, 768)
    F1 = (jnp.dot(_PF1, wf1p).T
          + bf1p.reshape(64, 1) * _BH1[None, :])
    F2 = jnp.transpose(wf2p[:, :16])
    bf2v = jnp.broadcast_to(bf2p[0, :16].reshape(16, 1), (16, LB))

    flops = G * 2 * LB * (6 * 960 * 256 + 4 * 320 * 768 + 64 * 400 + 16 * 64)
    bytes_accessed = 4 * (Npad * (28 * 32 + 10)
                          + 960 * 256 + 320 * 768 + 64 * 400 + 16 * 64)
    out = pl.pallas_call(
        _lenet_kernel,
        out_shape=jax.ShapeDtypeStruct((Npad, 10), jnp.float32),
        grid_spec=pltpu.PrefetchScalarGridSpec(
            num_scalar_prefetch=0,
            grid=(G,),
            in_specs=[
                pl.BlockSpec((28, 32, LB), lambda i: (0, 0, i)),
                pl.BlockSpec((960, 256), lambda i: (0, 0)),
                pl.BlockSpec((320, 768), lambda i: (0, 0)),
                pl.BlockSpec((64, 400), lambda i: (0, 0)),
                pl.BlockSpec((16, 64), lambda i: (0, 0)),
                pl.BlockSpec((16, LB), lambda i: (0, 0)),
            ],
            out_specs=pl.BlockSpec((LB, 10), lambda i: (i, 0)),
            scratch_shapes=[
                pltpu.VMEM((12, 128, LB), jnp.float32),
                pltpu.VMEM((5, 80, LB), jnp.float32),
            ],
        ),
        compiler_params=pltpu.CompilerParams(
            dimension_semantics=("parallel",),
            vmem_limit_bytes=32 * 1024 * 1024,
        ),
        cost_estimate=pl.CostEstimate(
            flops=flops, transcendentals=N * 17,
            bytes_accessed=bytes_accessed),
    )(xp, W1, W2, F1, F2, bf2v)
    return out[:N] if Npad != N else out
